# Initial kernel scaffold; baseline (speedup 1.0000x reference)
import jax
import jax.numpy as jnp
from jax.experimental import pallas as pl


def kernel(z, cdf_params, training):
    raise NotImplementedError("write your pallas kernel here")


# speedup vs baseline: 134.2036x; 134.2036x over previous
import functools

import jax
import jax.numpy as jnp
import numpy as np
from jax import lax
from jax.experimental import pallas as pl
from jax.experimental.pallas import tpu as pltpu
from jax.experimental.pallas import tpu_sc as plsc

_NC = 2
_NS = 16
_NW = _NC * _NS
_LANES = 16

_BOUND = 10.0
_L = 64
_MAGIC = 12582912.0
_INV_BIN_W = float(np.float32(1.0) / np.float32(20.0 / _L))


def _bits_table_body(cdf_ref, out_ref):
    row = cdf_ref[0:1, :]
    m = jnp.max(row, axis=1, keepdims=True)
    e = jnp.exp(row - m)
    p = e / jnp.sum(e, axis=1, keepdims=True)
    out_ref[...] = -jnp.log2(p + 1e-9)


def _final_sum_body(ps_ref, out_ref):
    out_ref[0, 0] = jnp.sum(ps_ref[...])


def _sc_body(z_hbm, tbl_hbm, zq_hbm, psum_hbm, buf, tbl, accv, n_per_tile):
    cid = lax.axis_index("c")
    sid = lax.axis_index("s")
    wid = sid * _NC + cid
    base = wid * n_per_tile

    pltpu.sync_copy(tbl_hbm, tbl)
    pltpu.sync_copy(z_hbm.at[pl.ds(base, n_per_tile)], buf)

    def body(i, acc):
        v = buf[pl.ds(i * _LANES, _LANES)]
        zc = jnp.minimum(jnp.maximum(v, -_BOUND), _BOUND)
        zq = (zc + _MAGIC) - _MAGIC
        buf[pl.ds(i * _LANES, _LANES)] = zq
        t = (zq + _BOUND) * _INV_BIN_W
        idx = jnp.minimum(t.astype(jnp.int32), _L - 1)
        return acc + plsc.load_gather(tbl, [idx])

    acc = lax.fori_loop(0, n_per_tile // _LANES, body,
                        jnp.zeros((_LANES,), jnp.float32))
    accv[...] = acc
    pltpu.sync_copy(buf, zq_hbm.at[pl.ds(base, n_per_tile)])
    pltpu.sync_copy(accv, psum_hbm.at[wid])


@jax.jit
def _entropy_model(z, cdf_params):
    n = z.size
    n_per_tile = n // _NW

    bits_table = pl.pallas_call(
        _bits_table_body,
        out_shape=jax.ShapeDtypeStruct((1, _L), jnp.float32),
    )(cdf_params)

    mesh = plsc.VectorSubcoreMesh(core_axis_name="c", subcore_axis_name="s")
    zq_flat, psums = pl.kernel(
        functools.partial(_sc_body, n_per_tile=n_per_tile),
        mesh=mesh,
        out_type=[
            jax.ShapeDtypeStruct((n,), jnp.float32),
            jax.ShapeDtypeStruct((_NW, _LANES), jnp.float32),
        ],
        scratch_types=[
            pltpu.VMEM((n_per_tile,), jnp.float32),
            pltpu.VMEM((_L,), jnp.float32),
            pltpu.VMEM((_LANES,), jnp.float32),
        ],
        compiler_params=pltpu.CompilerParams(needs_layout_passes=False),
    )(z.reshape(n), bits_table.reshape(_L))

    bits_sum = pl.pallas_call(
        _final_sum_body,
        out_shape=jax.ShapeDtypeStruct((1, 1), jnp.float32),
        out_specs=pl.BlockSpec(memory_space=pltpu.SMEM),
    )(psums)

    return bits_sum[0, 0], zq_flat.reshape(z.shape)


def kernel(z, cdf_params, training):
    return _entropy_model(z, cdf_params)


# speedup vs baseline: 154.1025x; 1.1483x over previous
import functools

import jax
import jax.numpy as jnp
import numpy as np
from jax import lax
from jax.experimental import pallas as pl
from jax.experimental.pallas import tpu as pltpu
from jax.experimental.pallas import tpu_sc as plsc

_NC = 2
_NS = 16
_NW = _NC * _NS
_LANES = 16

_BOUND = 10.0
_L = 64
_MAGIC = 12582912.0
_INV_BIN_W = float(np.float32(1.0) / np.float32(20.0 / _L))
_UNROLL = 8


def _bits_table_body(cdf_ref, out_ref):
    row = cdf_ref[0:1, :]
    m = jnp.max(row, axis=1, keepdims=True)
    e = jnp.exp(row - m)
    p = e / jnp.sum(e, axis=1, keepdims=True)
    out_ref[...] = -jnp.log2(p + 1e-9)


def _final_sum_body(ps_ref, out_ref):
    out_ref[0, 0] = jnp.sum(ps_ref[...])


def _sc_body(z_hbm, tbl_hbm, zq_hbm, psum_hbm, buf, tbl, accv, n_per_tile):
    cid = lax.axis_index("c")
    sid = lax.axis_index("s")
    wid = sid * _NC + cid
    base = wid * n_per_tile

    pltpu.sync_copy(tbl_hbm, tbl)
    pltpu.sync_copy(z_hbm.at[pl.ds(base, n_per_tile)], buf)

    zero = jnp.zeros((_LANES,), jnp.float32)

    @plsc.parallel_loop(0, n_per_tile, step=_LANES * _UNROLL,
                        carry=(zero,) * _UNROLL)
    def accs(off, accs_in):
        accs_out = []
        for u in range(_UNROLL):
            v = buf[pl.ds(off + u * _LANES, _LANES)]
            zc = jnp.minimum(jnp.maximum(v, -_BOUND), _BOUND)
            zq = (zc + _MAGIC) - _MAGIC
            buf[pl.ds(off + u * _LANES, _LANES)] = zq
            t = (zq + _BOUND) * _INV_BIN_W
            idx = jnp.minimum(t.astype(jnp.int32), _L - 1)
            accs_out.append(accs_in[u] + plsc.load_gather(tbl, [idx]))
        return tuple(accs_out)

    acc = accs[0]
    for u in range(1, _UNROLL):
        acc = acc + accs[u]
    accv[...] = acc
    pltpu.sync_copy(buf, zq_hbm.at[pl.ds(base, n_per_tile)])
    pltpu.sync_copy(accv, psum_hbm.at[wid])


@jax.jit
def _entropy_model(z, cdf_params):
    n = z.size
    n_per_tile = n // _NW

    bits_table = pl.pallas_call(
        _bits_table_body,
        out_shape=jax.ShapeDtypeStruct((1, _L), jnp.float32),
    )(cdf_params)

    mesh = plsc.VectorSubcoreMesh(core_axis_name="c", subcore_axis_name="s")
    zq_flat, psums = pl.kernel(
        functools.partial(_sc_body, n_per_tile=n_per_tile),
        mesh=mesh,
        out_type=[
            jax.ShapeDtypeStruct((n,), jnp.float32),
            jax.ShapeDtypeStruct((_NW, _LANES), jnp.float32),
        ],
        scratch_types=[
            pltpu.VMEM((n_per_tile,), jnp.float32),
            pltpu.VMEM((_L,), jnp.float32),
            pltpu.VMEM((_LANES,), jnp.float32),
        ],
        compiler_params=pltpu.CompilerParams(needs_layout_passes=False),
    )(z.reshape(n), bits_table.reshape(_L))

    bits_sum = pl.pallas_call(
        _final_sum_body,
        out_shape=jax.ShapeDtypeStruct((1, 1), jnp.float32),
        out_specs=pl.BlockSpec(memory_space=pltpu.SMEM),
    )(psums)

    return bits_sum[0, 0], zq_flat.reshape(z.shape)


def kernel(z, cdf_params, training):
    return _entropy_model(z, cdf_params)


# speedup vs baseline: 156.6878x; 1.0168x over previous
import functools

import jax
import jax.numpy as jnp
import numpy as np
from jax import lax
from jax.experimental import pallas as pl
from jax.experimental.pallas import tpu as pltpu
from jax.experimental.pallas import tpu_sc as plsc

_NC = 2
_NS = 16
_NW = _NC * _NS
_LANES = 16

_BOUND = 10.0
_L = 64
_MAGIC = 12582912.0
_INV_BIN_W = float(np.float32(1.0) / np.float32(20.0 / _L))
_UNROLL = 8

_LOG2_POLY = (
    -0.025123203173279762,
    0.2700374722480774,
    -1.247962474822998,
    3.24946665763855,
    -5.301709175109863,
    6.089895725250244,
    -3.0346028804779053,
)


def _log2_vec(x):
    i = plsc.bitcast(x, jnp.int32)
    e = (lax.shift_right_logical(i, 23) - 127).astype(jnp.float32)
    m = plsc.bitcast((i & 0x7FFFFF) | 0x3F800000, jnp.float32)
    p = jnp.full((_LANES,), _LOG2_POLY[0], jnp.float32)
    for c in _LOG2_POLY[1:]:
        p = p * m + c
    return e + p


def _final_sum_body(ps_ref, out_ref):
    out_ref[0, 0] = jnp.sum(ps_ref[...])


def _sc_body(z_hbm, cdf_hbm, zq_hbm, psum_hbm, buf, cdfv, tbl, accv,
             n_per_tile):
    cid = lax.axis_index("c")
    sid = lax.axis_index("s")
    wid = sid * _NC + cid
    base = wid * n_per_tile

    pltpu.sync_copy(cdf_hbm, cdfv)
    pltpu.sync_copy(z_hbm.at[pl.ds(base, n_per_tile)], buf)

    nv = _L // _LANES
    rows = [cdfv[pl.ds(u * _LANES, _LANES)] for u in range(nv)]
    m = rows[0]
    for r in rows[1:]:
        m = jnp.maximum(m, r)
    mx = jnp.max(m)
    exps = [jnp.exp(r - mx) for r in rows]
    s = exps[0]
    for e in exps[1:]:
        s = s + e
    inv = 1.0 / jnp.full((_LANES,), jnp.sum(s), jnp.float32)
    for u in range(nv):
        tbl[pl.ds(u * _LANES, _LANES)] = -_log2_vec(exps[u] * inv + 1e-9)

    zero = jnp.zeros((_LANES,), jnp.float32)

    @plsc.parallel_loop(0, n_per_tile, step=_LANES * _UNROLL,
                        carry=(zero,) * _UNROLL)
    def accs(off, accs_in):
        accs_out = []
        for u in range(_UNROLL):
            v = buf[pl.ds(off + u * _LANES, _LANES)]
            zc = jnp.minimum(jnp.maximum(v, -_BOUND), _BOUND)
            zq = (zc + _MAGIC) - _MAGIC
            buf[pl.ds(off + u * _LANES, _LANES)] = zq
            t = (zq + _BOUND) * _INV_BIN_W
            idx = jnp.minimum(t.astype(jnp.int32), _L - 1)
            accs_out.append(accs_in[u] + plsc.load_gather(tbl, [idx]))
        return tuple(accs_out)

    acc = accs[0]
    for u in range(1, _UNROLL):
        acc = acc + accs[u]
    accv[...] = acc
    pltpu.sync_copy(buf, zq_hbm.at[pl.ds(base, n_per_tile)])
    pltpu.sync_copy(accv, psum_hbm.at[wid])


@jax.jit
def _entropy_model(z, cdf_params):
    n = z.size
    n_per_tile = n // _NW

    mesh = plsc.VectorSubcoreMesh(core_axis_name="c", subcore_axis_name="s")
    zq_flat, psums = pl.kernel(
        functools.partial(_sc_body, n_per_tile=n_per_tile),
        mesh=mesh,
        out_type=[
            jax.ShapeDtypeStruct((n,), jnp.float32),
            jax.ShapeDtypeStruct((_NW, _LANES), jnp.float32),
        ],
        scratch_types=[
            pltpu.VMEM((n_per_tile,), jnp.float32),
            pltpu.VMEM((_L,), jnp.float32),
            pltpu.VMEM((_L,), jnp.float32),
            pltpu.VMEM((_LANES,), jnp.float32),
        ],
        compiler_params=pltpu.CompilerParams(needs_layout_passes=False),
    )(z.reshape(n), cdf_params[0])

    bits_sum = pl.pallas_call(
        _final_sum_body,
        out_shape=jax.ShapeDtypeStruct((1, 1), jnp.float32),
        out_specs=pl.BlockSpec(memory_space=pltpu.SMEM),
    )(psums)

    return bits_sum[0, 0], zq_flat.reshape(z.shape)


def kernel(z, cdf_params, training):
    return _entropy_model(z, cdf_params)


# speedup vs baseline: 208.7716x; 1.3324x over previous
import functools

import jax
import jax.numpy as jnp
import numpy as np
from jax import lax
from jax.experimental import pallas as pl
from jax.experimental.pallas import tpu as pltpu
from jax.experimental.pallas import tpu_sc as plsc

_NC = 2
_NS = 16
_NW = _NC * _NS
_LANES = 16

_BOUND = 10.0
_L = 64
_INV_BIN_W = float(np.float32(1.0) / np.float32(20.0 / _L))
_UNROLL = 8

_LOG2_POLY = (
    -0.025123203173279762,
    0.2700374722480774,
    -1.247962474822998,
    3.24946665763855,
    -5.301709175109863,
    6.089895725250244,
    -3.0346028804779053,
)


def _log2_vec(x):
    i = plsc.bitcast(x, jnp.int32)
    e = (lax.shift_right_logical(i, 23) - 127).astype(jnp.float32)
    m = plsc.bitcast((i & 0x7FFFFF) | 0x3F800000, jnp.float32)
    p = jnp.full((_LANES,), _LOG2_POLY[0], jnp.float32)
    for c in _LOG2_POLY[1:]:
        p = p * m + c
    return e + p


def _quant_body(z_ref, zq_ref, idx_ref):
    v = z_ref[0]
    zc = jnp.minimum(jnp.maximum(v, -_BOUND), _BOUND)
    zq = jnp.round(zc)
    zq_ref[0] = zq
    t = (zq + _BOUND) * _INV_BIN_W
    idx = jnp.minimum(t.astype(jnp.int32), _L - 1)
    idx_c = jnp.concatenate([idx[:, 8 * k:8 * (k + 1), :] for k in range(4)],
                            axis=-1)
    idx_ref[...] = idx_c.reshape(1024, 128)


def _final_sum_body(ps_ref, out_ref):
    out_ref[0, 0] = jnp.sum(ps_ref[...])


def _sc_body(idx_hbm, cdf_hbm, psum_hbm, buf, cdfv, tbl, accv, n_per_tile):
    cid = lax.axis_index("c")
    sid = lax.axis_index("s")
    wid = sid * _NC + cid
    base = wid * n_per_tile

    pltpu.sync_copy(cdf_hbm, cdfv)
    pltpu.sync_copy(idx_hbm.at[pl.ds(base, n_per_tile)], buf)

    nv = _L // _LANES
    rows = [cdfv[pl.ds(u * _LANES, _LANES)] for u in range(nv)]
    m = rows[0]
    for r in rows[1:]:
        m = jnp.maximum(m, r)
    mx = jnp.max(m)
    exps = [jnp.exp(r - mx) for r in rows]
    s = exps[0]
    for e in exps[1:]:
        s = s + e
    inv = 1.0 / jnp.full((_LANES,), jnp.sum(s), jnp.float32)
    for u in range(nv):
        tbl[pl.ds(u * _LANES, _LANES)] = -_log2_vec(exps[u] * inv + 1e-9)

    zero = jnp.zeros((_LANES,), jnp.float32)

    @plsc.parallel_loop(0, n_per_tile, step=_LANES * _UNROLL,
                        carry=(zero,) * _UNROLL)
    def accs(off, accs_in):
        accs_out = []
        for u in range(_UNROLL):
            idx = buf[pl.ds(off + u * _LANES, _LANES)]
            accs_out.append(accs_in[u] + plsc.load_gather(tbl, [idx]))
        return tuple(accs_out)

    acc = accs[0]
    for u in range(1, _UNROLL):
        acc = acc + accs[u]
    accv[...] = acc
    pltpu.sync_copy(accv, psum_hbm.at[wid])


@jax.jit
def _entropy_model(z, cdf_params):
    n = z.size
    n_per_tile = n // _NW

    zq, idx2d = pl.pallas_call(
        _quant_body,
        grid=(z.shape[0],),
        in_specs=[pl.BlockSpec((1,) + z.shape[1:], lambda i: (i, 0, 0, 0))],
        out_specs=[
            pl.BlockSpec((1,) + z.shape[1:], lambda i: (i, 0, 0, 0)),
            pl.BlockSpec((n // z.shape[0] // 128, 128), lambda i: (i, 0)),
        ],
        out_shape=[
            jax.ShapeDtypeStruct(z.shape, jnp.float32),
            jax.ShapeDtypeStruct((n // 128, 128), jnp.int32),
        ],
    )(z)

    mesh = plsc.VectorSubcoreMesh(core_axis_name="c", subcore_axis_name="s")
    psums = pl.kernel(
        functools.partial(_sc_body, n_per_tile=n_per_tile),
        mesh=mesh,
        out_type=jax.ShapeDtypeStruct((_NW, _LANES), jnp.float32),
        scratch_types=[
            pltpu.VMEM((n_per_tile,), jnp.int32),
            pltpu.VMEM((_L,), jnp.float32),
            pltpu.VMEM((_L,), jnp.float32),
            pltpu.VMEM((_LANES,), jnp.float32),
        ],
        compiler_params=pltpu.CompilerParams(needs_layout_passes=False),
    )(idx2d.reshape(n), cdf_params[0])

    bits_sum = pl.pallas_call(
        _final_sum_body,
        out_shape=jax.ShapeDtypeStruct((1, 1), jnp.float32),
        out_specs=pl.BlockSpec(memory_space=pltpu.SMEM),
    )(psums)

    return bits_sum[0, 0], zq


def kernel(z, cdf_params, training):
    return _entropy_model(z, cdf_params)
